# baseline (speedup 1.0000x reference)
import functools

import jax
import jax.numpy as jnp
from jax import lax
from jax.experimental import pallas as pl
from jax.experimental.pallas import tpu as pltpu
from jax.experimental.pallas import tpu_sc as plsc

N = 10000
F = 128
NS = 2
RB = 200
NSTEP1 = N // (NS * RB)
NT = 8000
NSTEP2 = NT // (NS * RB)
SC_ROWS = 2048
SC_BASE = N - SC_ROWS
RPT = SC_ROWS // 32
L = 16
NCHUNK = N // L


def _p1_body(adjA_ref, adjB_ref, x_ref, W1_ref, b1_ref, W2_ref, Wc_ref,
             u_ref, s1_ref, v_ref):
    i = pl.program_id(0)

    @pl.when(i == 0)
    def _init():
        s1_ref[...] = jnp.dot(x_ref[...], W1_ref[...],
                              preferred_element_type=jnp.float32)
        v_ref[...] = jnp.dot(W2_ref[...], Wc_ref[...],
                             preferred_element_type=jnp.float32)

    for s, a_ref in enumerate((adjA_ref, adjB_ref)):
        z = jnp.dot(a_ref[...], s1_ref[...],
                    preferred_element_type=jnp.float32)
        z = jnp.maximum(z + b1_ref[...], 0.0)
        u_ref[s * RB:(s + 1) * RB, :] = jnp.dot(
            z, v_ref[...], preferred_element_type=jnp.float32)


def _p2_body(adjA_ref, adjB_ref, u_ref, c_ref, out_ref):
    for s, a_ref in enumerate((adjA_ref, adjB_ref)):
        out_ref[s * RB:(s + 1) * RB, :] = jnp.dot(
            a_ref[...], u_ref[...],
            preferred_element_type=jnp.float32) + c_ref[0, 0]


def _sc_body(adj_hbm, u_hbm, out_hbm, ubuf, rb0, rb1, tot, semu, sem0, sem1):
    nc = 2
    wid = lax.axis_index("s") * nc + lax.axis_index("c")
    base = SC_BASE + wid * RPT

    pltpu.async_copy(u_hbm, ubuf, semu).wait()
    pltpu.async_copy(adj_hbm.at[base], rb0, sem0)
    pltpu.async_copy(adj_hbm.at[base + 1], rb1, sem1)

    def _row_partial(rbuf):
        def chunk(ci, acc):
            a = rbuf[pl.ds(ci * L, L)]
            uu = ubuf[pl.ds(ci * L, L)]
            return acc + a * uu
        return lax.fori_loop(0, NCHUNK, chunk, jnp.zeros((L,), jnp.float32))

    def pair(g, carry):
        row = base + 2 * g
        pltpu.make_async_copy(adj_hbm.at[row], rb0, sem0).wait()
        acc0 = _row_partial(rb0)

        @pl.when(g + 1 < RPT // 2)
        def _n0():
            pltpu.async_copy(adj_hbm.at[row + 2], rb0, sem0)

        tot[pl.ds((2 * g) * L, L)] = acc0

        pltpu.make_async_copy(adj_hbm.at[row + 1], rb1, sem1).wait()
        acc1 = _row_partial(rb1)

        @pl.when(g + 1 < RPT // 2)
        def _n1():
            pltpu.async_copy(adj_hbm.at[row + 3], rb1, sem1)

        tot[pl.ds((2 * g + 1) * L, L)] = acc1
        return carry

    lax.fori_loop(0, RPT // 2, pair, 0)
    pltpu.sync_copy(tot, out_hbm.at[pl.ds(wid * RPT * L, RPT * L)])


def _sc_reduce_body(p_ref, c_ref, out_ref):
    out_ref[...] = (jnp.sum(p_ref[...], axis=1, keepdims=True)
                    + c_ref[0, 0])


def kernel(adj, x, W1, b1, W2, b2, Wc, bc, We1, be1, We2, be2, Wfc, bfc):
    del We1, be1, We2, be2, Wfc, bfc
    b1_2d = b1.reshape(1, F)

    adj_specs = [
        pl.BlockSpec((RB, N), lambda i, s=s: (NS * i + s, 0))
        for s in range(NS)
    ]
    u = pl.pallas_call(
        _p1_body,
        grid=(NSTEP1,),
        in_specs=adj_specs + [
            pl.BlockSpec((N, F), lambda i: (0, 0)),
            pl.BlockSpec((F, F), lambda i: (0, 0)),
            pl.BlockSpec((1, F), lambda i: (0, 0)),
            pl.BlockSpec((F, F), lambda i: (0, 0)),
            pl.BlockSpec((F, 1), lambda i: (0, 0)),
        ],
        out_specs=pl.BlockSpec((NS * RB, 1), lambda i: (i, 0)),
        out_shape=jax.ShapeDtypeStruct((N, 1), jnp.float32),
        scratch_shapes=[
            pltpu.VMEM((N, F), jnp.float32),
            pltpu.VMEM((F, 1), jnp.float32),
        ],
    )(adj, adj, x, W1, b1_2d, W2, Wc)

    c = (b2.reshape(1, F) @ Wc + bc).reshape(1, 1)

    sc_fn = functools.partial(
        pl.kernel,
        out_type=jax.ShapeDtypeStruct((SC_ROWS * L,), jnp.float32),
        mesh=plsc.VectorSubcoreMesh(core_axis_name="c", subcore_axis_name="s"),
        scratch_types=[
            pltpu.VMEM((N,), jnp.float32),
            pltpu.VMEM((N,), jnp.float32),
            pltpu.VMEM((N,), jnp.float32),
            pltpu.VMEM((RPT * L,), jnp.float32),
            pltpu.SemaphoreType.DMA,
            pltpu.SemaphoreType.DMA,
            pltpu.SemaphoreType.DMA,
        ],
    )(_sc_body)
    sc_partials = sc_fn(adj, u.reshape(N))

    sc_out = pl.pallas_call(
        _sc_reduce_body,
        grid=(1,),
        in_specs=[
            pl.BlockSpec((SC_ROWS, L), lambda i: (0, 0)),
            pl.BlockSpec((1, 1), lambda i: (0, 0), memory_space=pltpu.SMEM),
        ],
        out_specs=pl.BlockSpec((SC_ROWS, 1), lambda i: (0, 0)),
        out_shape=jax.ShapeDtypeStruct((SC_ROWS, 1), jnp.float32),
    )(sc_partials.reshape(SC_ROWS, L), c)

    tc_out = pl.pallas_call(
        _p2_body,
        grid=(NSTEP2,),
        in_specs=adj_specs + [
            pl.BlockSpec((N, 1), lambda i: (0, 0)),
            pl.BlockSpec((1, 1), lambda i: (0, 0), memory_space=pltpu.SMEM),
        ],
        out_specs=pl.BlockSpec((NS * RB, 1), lambda i: (i, 0)),
        out_shape=jax.ShapeDtypeStruct((NT, 1), jnp.float32),
    )(adj, adj, u, c)

    return jnp.concatenate([tc_out[:SC_BASE], sc_out], axis=0)


# speedup vs baseline: 1.1182x; 1.1182x over previous
import functools

import jax
import jax.numpy as jnp
from jax import lax
from jax.experimental import pallas as pl
from jax.experimental.pallas import tpu as pltpu
from jax.experimental.pallas import tpu_sc as plsc

N = 10000
F = 128
NS = 2
RB = 200
NSTEP1 = N // (NS * RB)
NT = 9200
NSTEP2 = NT // (NS * RB)
SC_ROWS = 1024
SC_BASE = N - SC_ROWS
RPT = SC_ROWS // 32
L = 16
NCHUNK = N // L
UNROLL = 5


def _p1_body(adjA_ref, adjB_ref, x_ref, W1_ref, b1_ref, W2_ref, Wc_ref,
             u_ref, s1_ref, v_ref):
    i = pl.program_id(0)

    @pl.when(i == 0)
    def _init():
        s1_ref[...] = jnp.dot(x_ref[...], W1_ref[...],
                              preferred_element_type=jnp.float32)
        v_ref[...] = jnp.dot(W2_ref[...], Wc_ref[...],
                             preferred_element_type=jnp.float32)

    for s, a_ref in enumerate((adjA_ref, adjB_ref)):
        z = jnp.dot(a_ref[...], s1_ref[...],
                    preferred_element_type=jnp.float32)
        z = jnp.maximum(z + b1_ref[...], 0.0)
        u_ref[s * RB:(s + 1) * RB, :] = jnp.dot(
            z, v_ref[...], preferred_element_type=jnp.float32)


def _p2_body(adjA_ref, adjB_ref, u_ref, c_ref, out_ref):
    for s, a_ref in enumerate((adjA_ref, adjB_ref)):
        out_ref[s * RB:(s + 1) * RB, :] = jnp.dot(
            a_ref[...], u_ref[...],
            preferred_element_type=jnp.float32) + c_ref[0, 0]


def _sc_body(adj_hbm, u_hbm, out_hbm, ubuf, rb0, rb1, tot, semu, sem0, sem1):
    nc = 2
    wid = lax.axis_index("s") * nc + lax.axis_index("c")
    base = SC_BASE + wid * RPT

    pltpu.async_copy(u_hbm, ubuf, semu).wait()
    pltpu.async_copy(adj_hbm.at[base], rb0, sem0)
    pltpu.async_copy(adj_hbm.at[base + 1], rb1, sem1)

    def _row_partial(rbuf):
        def chunk(ci, acc):
            for k in range(UNROLL):
                off = (ci * UNROLL + k) * L
                acc = acc + rbuf[pl.ds(off, L)] * ubuf[pl.ds(off, L)]
            return acc
        return lax.fori_loop(0, NCHUNK // UNROLL, chunk,
                             jnp.zeros((L,), jnp.float32))

    def pair(g, carry):
        row = base + 2 * g
        pltpu.make_async_copy(adj_hbm.at[row], rb0, sem0).wait()
        acc0 = _row_partial(rb0)

        @pl.when(g + 1 < RPT // 2)
        def _n0():
            pltpu.async_copy(adj_hbm.at[row + 2], rb0, sem0)

        tot[pl.ds((2 * g) * L, L)] = acc0

        pltpu.make_async_copy(adj_hbm.at[row + 1], rb1, sem1).wait()
        acc1 = _row_partial(rb1)

        @pl.when(g + 1 < RPT // 2)
        def _n1():
            pltpu.async_copy(adj_hbm.at[row + 3], rb1, sem1)

        tot[pl.ds((2 * g + 1) * L, L)] = acc1
        return carry

    lax.fori_loop(0, RPT // 2, pair, 0)
    pltpu.sync_copy(tot, out_hbm.at[pl.ds(wid * RPT * L, RPT * L)])


def _sc_reduce_body(p_ref, c_ref, out_ref):
    out_ref[...] = (jnp.sum(p_ref[...], axis=1, keepdims=True)
                    + c_ref[0, 0])


def kernel(adj, x, W1, b1, W2, b2, Wc, bc, We1, be1, We2, be2, Wfc, bfc):
    del We1, be1, We2, be2, Wfc, bfc
    b1_2d = b1.reshape(1, F)

    adj_specs = [
        pl.BlockSpec((RB, N), lambda i, s=s: (NS * i + s, 0))
        for s in range(NS)
    ]
    u = pl.pallas_call(
        _p1_body,
        grid=(NSTEP1,),
        in_specs=adj_specs + [
            pl.BlockSpec((N, F), lambda i: (0, 0)),
            pl.BlockSpec((F, F), lambda i: (0, 0)),
            pl.BlockSpec((1, F), lambda i: (0, 0)),
            pl.BlockSpec((F, F), lambda i: (0, 0)),
            pl.BlockSpec((F, 1), lambda i: (0, 0)),
        ],
        out_specs=pl.BlockSpec((NS * RB, 1), lambda i: (i, 0)),
        out_shape=jax.ShapeDtypeStruct((N, 1), jnp.float32),
        scratch_shapes=[
            pltpu.VMEM((N, F), jnp.float32),
            pltpu.VMEM((F, 1), jnp.float32),
        ],
    )(adj, adj, x, W1, b1_2d, W2, Wc)

    c = (b2.reshape(1, F) @ Wc + bc).reshape(1, 1)

    sc_fn = functools.partial(
        pl.kernel,
        out_type=jax.ShapeDtypeStruct((SC_ROWS * L,), jnp.float32),
        mesh=plsc.VectorSubcoreMesh(core_axis_name="c", subcore_axis_name="s"),
        scratch_types=[
            pltpu.VMEM((N,), jnp.float32),
            pltpu.VMEM((N,), jnp.float32),
            pltpu.VMEM((N,), jnp.float32),
            pltpu.VMEM((RPT * L,), jnp.float32),
            pltpu.SemaphoreType.DMA,
            pltpu.SemaphoreType.DMA,
            pltpu.SemaphoreType.DMA,
        ],
    )(_sc_body)
    sc_partials = sc_fn(adj, u.reshape(N))

    sc_out = pl.pallas_call(
        _sc_reduce_body,
        grid=(1,),
        in_specs=[
            pl.BlockSpec((SC_ROWS, L), lambda i: (0, 0)),
            pl.BlockSpec((1, 1), lambda i: (0, 0), memory_space=pltpu.SMEM),
        ],
        out_specs=pl.BlockSpec((SC_ROWS, 1), lambda i: (0, 0)),
        out_shape=jax.ShapeDtypeStruct((SC_ROWS, 1), jnp.float32),
    )(sc_partials.reshape(SC_ROWS, L), c)

    tc_out = pl.pallas_call(
        _p2_body,
        grid=(NSTEP2,),
        in_specs=adj_specs + [
            pl.BlockSpec((N, 1), lambda i: (0, 0)),
            pl.BlockSpec((1, 1), lambda i: (0, 0), memory_space=pltpu.SMEM),
        ],
        out_specs=pl.BlockSpec((NS * RB, 1), lambda i: (i, 0)),
        out_shape=jax.ShapeDtypeStruct((NT, 1), jnp.float32),
    )(adj, adj, u, c)

    return jnp.concatenate([tc_out[:SC_BASE], sc_out], axis=0)


# speedup vs baseline: 1.3358x; 1.1946x over previous
import jax
import jax.numpy as jnp
from jax.experimental import pallas as pl
from jax.experimental.pallas import tpu as pltpu

N = 10000
F = 128
RB = 200
NSTEP = N // (2 * RB)


def _body(adjA_ref, adjB_ref, x_ref, W1_ref, b1_ref, W2_ref, b2_ref,
          Wc_ref, bc_ref, out_ref, u_ref, s1_ref, v_ref):
    i = pl.program_id(0)

    @pl.when(i == 0)
    def _init():
        s1_ref[...] = jnp.dot(x_ref[...], W1_ref[...],
                              preferred_element_type=jnp.float32)
        v_ref[...] = jnp.dot(W2_ref[...], Wc_ref[...],
                             preferred_element_type=jnp.float32)

    @pl.when(i < NSTEP)
    def _phase1():
        zA = jnp.dot(adjA_ref[...], s1_ref[...],
                     preferred_element_type=jnp.float32)
        zB = jnp.dot(adjB_ref[...], s1_ref[...],
                     preferred_element_type=jnp.float32)
        zA = jnp.maximum(zA + b1_ref[...], 0.0)
        zB = jnp.maximum(zB + b1_ref[...], 0.0)
        blk = i * 2 * RB
        u_ref[pl.ds(blk, RB), :] = jnp.dot(
            zA, v_ref[...], preferred_element_type=jnp.float32)
        u_ref[pl.ds(blk + RB, RB), :] = jnp.dot(
            zB, v_ref[...], preferred_element_type=jnp.float32)

    @pl.when(i >= NSTEP)
    def _phase2():
        c = jnp.dot(b2_ref[...], Wc_ref[...],
                    preferred_element_type=jnp.float32) + bc_ref[...]
        out_ref[:RB, :] = jnp.dot(adjA_ref[...], u_ref[...],
                                  preferred_element_type=jnp.float32) + c
        out_ref[RB:, :] = jnp.dot(adjB_ref[...], u_ref[...],
                                  preferred_element_type=jnp.float32) + c


def kernel(adj, x, W1, b1, W2, b2, Wc, bc, We1, be1, We2, be2, Wfc, bfc):
    del We1, be1, We2, be2, Wfc, bfc
    b1_2d = b1.reshape(1, F)
    b2_2d = b2.reshape(1, F)
    bc_2d = bc.reshape(1, 1)

    label = pl.pallas_call(
        _body,
        grid=(2 * NSTEP,),
        in_specs=[
            pl.BlockSpec((RB, N), lambda i: (2 * (i % NSTEP), 0)),
            pl.BlockSpec((RB, N), lambda i: (2 * (i % NSTEP) + 1, 0)),
            pl.BlockSpec((N, F), lambda i: (0, 0)),
            pl.BlockSpec((F, F), lambda i: (0, 0)),
            pl.BlockSpec((1, F), lambda i: (0, 0)),
            pl.BlockSpec((F, F), lambda i: (0, 0)),
            pl.BlockSpec((1, F), lambda i: (0, 0)),
            pl.BlockSpec((F, 1), lambda i: (0, 0)),
            pl.BlockSpec((1, 1), lambda i: (0, 0)),
        ],
        out_specs=pl.BlockSpec(
            (2 * RB, 1), lambda i: (jnp.where(i < NSTEP, 0, i - NSTEP), 0)),
        out_shape=jax.ShapeDtypeStruct((N, 1), jnp.float32),
        scratch_shapes=[
            pltpu.VMEM((N, 1), jnp.float32),
            pltpu.VMEM((N, F), jnp.float32),
            pltpu.VMEM((F, 1), jnp.float32),
        ],
    )(adj, adj, x, W1, b1_2d, W2, b2_2d, Wc, bc_2d)
    return label
